# Initial kernel scaffold; baseline (speedup 1.0000x reference)
import functools

import jax
import jax.numpy as jnp
from jax import lax
from jax.experimental import pallas as pl
from jax.experimental.pallas import tpu as pltpu
from jax.experimental.pallas import tpu_sc as plsc

NC = 2
NS = 16
LN = 16
EB = 128


def _sc_mesh():
  return plsc.VectorSubcoreMesh(core_axis_name="c", subcore_axis_name="s")


def _make_deg_kernel(CH, NHP):
  HPT = NHP // NS

  @functools.partial(
      pl.kernel,
      mesh=_sc_mesh(),
      out_type=jax.ShapeDtypeStruct((NC, 2, NHP), jnp.float32),
      scratch_types=[
          pltpu.VMEM((CH, EB), jnp.int32),
          pltpu.VMEM((CH, EB), jnp.int32),
          pltpu.VMEM((EB,), jnp.float32),
          pltpu.VMEM((HPT,), jnp.float32),
          pltpu.VMEM_SHARED((NHP,), jnp.float32),
          pltpu.VMEM_SHARED((NHP,), jnp.float32),
      ],
  )
  def deg_kernel(src_hbm, dst_hbm, out_hbm, src_v, dst_v, ones_v, z_v,
                 ho_s, hi_s):
    c = lax.axis_index("c")
    s = lax.axis_index("s")
    wid = s * NC + c

    for k in range(EB // LN):
      ones_v[pl.ds(k * LN, LN)] = jnp.ones((LN,), jnp.float32)
    for k in range(HPT // LN):
      z_v[pl.ds(k * LN, LN)] = jnp.zeros((LN,), jnp.float32)

    pltpu.sync_copy(z_v, ho_s.at[pl.ds(s * HPT, HPT)])
    pltpu.sync_copy(z_v, hi_s.at[pl.ds(s * HPT, HPT)])
    plsc.subcore_barrier()

    pltpu.sync_copy(src_hbm.at[wid], src_v)
    pltpu.sync_copy(dst_hbm.at[wid], dst_v)

    def body(j, carry):
      pltpu.sync_copy(ones_v, ho_s.at[src_v.at[j]], add=True)
      pltpu.sync_copy(ones_v, hi_s.at[dst_v.at[j]], add=True)
      return carry

    lax.fori_loop(0, CH, body, 0)
    plsc.subcore_barrier()

    pltpu.sync_copy(ho_s.at[pl.ds(s * HPT, HPT)],
                    out_hbm.at[c, 0, pl.ds(s * HPT, HPT)])
    pltpu.sync_copy(hi_s.at[pl.ds(s * HPT, HPT)],
                    out_hbm.at[c, 1, pl.ds(s * HPT, HPT)])

  return deg_kernel


def _feat_body(h_ref, w_ref, dego_ref, o_ref):
  deg = dego_ref[:, 0] + dego_ref[:, 1]
  rs = lax.rsqrt(jnp.maximum(deg, 1.0))
  hs = h_ref[...] * rs[:, None]
  o_ref[...] = jnp.dot(hs, w_ref[...], preferred_element_type=jnp.float32)


def _feat_tc(h, W, dego_t, BR):
  N, D_in = h.shape
  D_out = W.shape[1]
  return pl.pallas_call(
      _feat_body,
      grid=(N // BR,),
      in_specs=[
          pl.BlockSpec((BR, D_in), lambda i: (i, 0)),
          pl.BlockSpec((D_in, D_out), lambda i: (0, 0)),
          pl.BlockSpec((BR, 2), lambda i: (i, 0)),
      ],
      out_specs=pl.BlockSpec((BR, D_out), lambda i: (i, 0)),
      out_shape=jax.ShapeDtypeStruct((N, D_out), jnp.float32),
  )(h, W, dego_t)


def _make_agg_kernel(NFP, D, CH, AGG):
  RPT = NFP // NS
  APT = AGG // NS

  @functools.partial(
      pl.kernel,
      mesh=_sc_mesh(),
      out_type=jax.ShapeDtypeStruct((NC, AGG, D), jnp.float32),
      scratch_types=[
          pltpu.VMEM((CH, EB), jnp.int32),
          pltpu.VMEM((CH, EB), jnp.int32),
          pltpu.VMEM((EB, D), jnp.float32),
          pltpu.VMEM((8, D), jnp.float32),
          pltpu.VMEM_SHARED((NFP, D), jnp.float32),
          pltpu.VMEM_SHARED((AGG, D), jnp.float32),
      ],
  )
  def agg_kernel(feat_hbm, src_hbm, ldst_hbm, out_hbm, src_v, ldst_v,
                 rows_v, z_v, feat_s, agg_s):
    c = lax.axis_index("c")
    s = lax.axis_index("s")
    wid = s * NC + c

    for r in range(8):
      for k in range(D // LN):
        z_v[r, pl.ds(k * LN, LN)] = jnp.zeros((LN,), jnp.float32)

    pltpu.sync_copy(feat_hbm.at[pl.ds(s * RPT, RPT)],
                    feat_s.at[pl.ds(s * RPT, RPT)])

    def zbody(k, carry):
      pltpu.sync_copy(z_v, agg_s.at[pl.ds(s * APT + k * 8, 8)])
      return carry

    lax.fori_loop(0, APT // 8, zbody, 0)
    plsc.subcore_barrier()

    pltpu.sync_copy(src_hbm.at[wid], src_v)
    pltpu.sync_copy(ldst_hbm.at[c, wid], ldst_v)

    def ebody(j, carry):
      pltpu.sync_copy(feat_s.at[src_v.at[j]], rows_v)
      pltpu.sync_copy(rows_v, agg_s.at[ldst_v.at[j]], add=True)
      return carry

    lax.fori_loop(0, CH, ebody, 0)
    plsc.subcore_barrier()

    pltpu.sync_copy(agg_s.at[pl.ds(s * APT, APT)],
                    out_hbm.at[c, pl.ds(s * APT, APT)])

  return agg_kernel


def _finish_body(agg_ref, degi_ref, b_ref, o_ref):
  deg = degi_ref[:, 0] + degi_ref[:, 1]
  rs = lax.rsqrt(jnp.maximum(deg, 1.0))
  o_ref[...] = jnp.maximum(agg_ref[0] * rs[:, None] + b_ref[...], 0.0)


def _finish_tc(aggp, degi_t, b2, N, D, BR):
  half = N // 2
  bpc = half // BR

  return pl.pallas_call(
      _finish_body,
      grid=(N // BR,),
      in_specs=[
          pl.BlockSpec((1, BR, D), lambda i: (i // bpc, i % bpc, 0)),
          pl.BlockSpec((BR, 2), lambda i: (i, 0)),
          pl.BlockSpec((1, D), lambda i: (0, 0)),
      ],
      out_specs=pl.BlockSpec((BR, D), lambda i: (i, 0)),
      out_shape=jax.ShapeDtypeStruct((N, D), jnp.float32),
  )(aggp, degi_t, b2)


def kernel(h, edge_index, W, b):
  N, D_in = h.shape
  D = W.shape[1]
  E = edge_index.shape[1]
  NW = NC * NS

  CH = -(-E // (NW * EB))
  E_pad = NW * CH * EB
  src = edge_index[0]
  dst = edge_index[1]
  pad = E_pad - E
  sent = jnp.full((pad,), N, jnp.int32)
  src_p = jnp.concatenate([src, sent]).reshape(NW, CH, EB)
  dst_flat = jnp.concatenate([dst, sent])
  dst_p = dst_flat.reshape(NW, CH, EB)

  half = N // 2
  AGG = -(-(half + 1) // (NS * 8)) * (NS * 8)
  in0 = dst_flat < half
  ldst0 = jnp.where(in0, dst_flat, half)
  in1 = jnp.logical_and(dst_flat >= half, dst_flat < N)
  ldst1 = jnp.where(in1, dst_flat - half, half)
  ldst_p = jnp.stack([ldst0, ldst1]).reshape(NC, NW, CH, EB)

  NHP = -(-(N + 1) // (NS * 8)) * (NS * 8)
  hp = _make_deg_kernel(CH, NHP)(src_p, dst_p)
  dego_t = hp[:, 0, :N].T
  degi_t = hp[:, 1, :N].T

  BR = 500
  feat = _feat_tc(h, W, dego_t, BR)

  NFP = -(-(N + 1) // NS) * NS
  feat_pad = jnp.zeros((NFP, D), jnp.float32).at[:N].set(feat)

  aggp = _make_agg_kernel(NFP, D, CH, AGG)(feat_pad, src_p, ldst_p)

  out = _finish_tc(aggp, degi_t, b.reshape(1, D), N, D, BR)
  return out


# speedup vs baseline: 6.3728x; 6.3728x over previous
import functools

import jax
import jax.numpy as jnp
from jax import lax
from jax.experimental import pallas as pl
from jax.experimental.pallas import tpu as pltpu
from jax.experimental.pallas import tpu_sc as plsc

NC = 2
NS = 16
LN = 16
EB = 128


def _sc_mesh():
  return plsc.VectorSubcoreMesh(core_axis_name="c", subcore_axis_name="s")


def _make_deg_kernel(CH, NHP):
  HPT = NHP // NS

  @functools.partial(
      pl.kernel,
      mesh=_sc_mesh(),
      out_type=jax.ShapeDtypeStruct((NC * 2 * NHP,), jnp.float32),
      scratch_types=[
          pltpu.VMEM((CH, EB), jnp.int32),
          pltpu.VMEM((CH, EB), jnp.int32),
          pltpu.VMEM((EB,), jnp.float32),
          pltpu.VMEM((HPT,), jnp.float32),
          pltpu.VMEM_SHARED((NHP,), jnp.float32),
          pltpu.VMEM_SHARED((NHP,), jnp.float32),
      ],
  )
  def deg_kernel(src_hbm, dst_hbm, out_hbm, src_v, dst_v, ones_v, z_v,
                 ho_s, hi_s):
    c = lax.axis_index("c")
    s = lax.axis_index("s")
    wid = s * NC + c

    for k in range(EB // LN):
      ones_v[pl.ds(k * LN, LN)] = jnp.ones((LN,), jnp.float32)
    for k in range(HPT // LN):
      z_v[pl.ds(k * LN, LN)] = jnp.zeros((LN,), jnp.float32)

    pltpu.sync_copy(z_v, ho_s.at[pl.ds(s * HPT, HPT)])
    pltpu.sync_copy(z_v, hi_s.at[pl.ds(s * HPT, HPT)])
    plsc.subcore_barrier()

    pltpu.sync_copy(src_hbm.at[wid], src_v)
    pltpu.sync_copy(dst_hbm.at[wid], dst_v)

    def body(j, carry):
      pltpu.sync_copy(ones_v, ho_s.at[src_v.at[j]], add=True)
      pltpu.sync_copy(ones_v, hi_s.at[dst_v.at[j]], add=True)
      return carry

    lax.fori_loop(0, CH, body, 0)
    plsc.subcore_barrier()

    base = c * (2 * NHP) + s * HPT
    pltpu.sync_copy(ho_s.at[pl.ds(s * HPT, HPT)], z_v)
    pltpu.sync_copy(z_v, out_hbm.at[pl.ds(base, HPT)])
    pltpu.sync_copy(hi_s.at[pl.ds(s * HPT, HPT)], z_v)
    pltpu.sync_copy(z_v, out_hbm.at[pl.ds(base + NHP, HPT)])

  return deg_kernel


def _feat_body(h_ref, w_ref, dego_ref, o_ref):
  deg = dego_ref[:, 0] + dego_ref[:, 1]
  rs = lax.rsqrt(jnp.maximum(deg, 1.0))
  hs = h_ref[...] * rs[:, None]
  o_ref[...] = jnp.dot(hs, w_ref[...], preferred_element_type=jnp.float32)


def _feat_tc(h, W, dego_t, BR):
  N, D_in = h.shape
  D_out = W.shape[1]
  return pl.pallas_call(
      _feat_body,
      grid=(N // BR,),
      in_specs=[
          pl.BlockSpec((BR, D_in), lambda i: (i, 0)),
          pl.BlockSpec((D_in, D_out), lambda i: (0, 0)),
          pl.BlockSpec((BR, 2), lambda i: (i, 0)),
      ],
      out_specs=pl.BlockSpec((BR, D_out), lambda i: (i, 0)),
      out_shape=jax.ShapeDtypeStruct((N, D_out), jnp.float32),
  )(h, W, dego_t)


def _make_agg_kernel(NFP, D, CH):
  APT = NFP // NS

  @functools.partial(
      pl.kernel,
      mesh=_sc_mesh(),
      out_type=jax.ShapeDtypeStruct((NC, NFP, D), jnp.float32),
      scratch_types=[
          pltpu.VMEM((CH, EB), jnp.int32),
          pltpu.VMEM((CH, EB), jnp.int32),
          pltpu.VMEM((EB, D), jnp.float32),
          pltpu.VMEM((8, D), jnp.float32),
          pltpu.VMEM_SHARED((NFP, D), jnp.float32),
      ],
  )
  def agg_kernel(feat_hbm, src_hbm, dst_hbm, out_hbm, src_v, dst_v,
                 rows_v, z_v, agg_s):
    c = lax.axis_index("c")
    s = lax.axis_index("s")
    slab = c * NS + s

    for r in range(8):
      for k in range(D // LN):
        z_v[r, pl.ds(k * LN, LN)] = jnp.zeros((LN,), jnp.float32)

    def zbody(k, carry):
      pltpu.sync_copy(z_v, agg_s.at[pl.ds(s * APT + k * 8, 8)])
      return carry

    lax.fori_loop(0, APT // 8, zbody, 0)
    plsc.subcore_barrier()

    pltpu.sync_copy(src_hbm.at[slab], src_v)
    pltpu.sync_copy(dst_hbm.at[slab], dst_v)

    def ebody(j, carry):
      pltpu.sync_copy(feat_hbm.at[src_v.at[j]], rows_v)
      pltpu.sync_copy(rows_v, agg_s.at[dst_v.at[j]], add=True)
      return carry

    lax.fori_loop(0, CH, ebody, 0)
    plsc.subcore_barrier()

    pltpu.sync_copy(agg_s.at[pl.ds(s * APT, APT)],
                    out_hbm.at[c, pl.ds(s * APT, APT)])

  return agg_kernel


def _finish_body(agg_ref, degi_ref, b_ref, o_ref):
  deg = degi_ref[:, 0] + degi_ref[:, 1]
  rs = lax.rsqrt(jnp.maximum(deg, 1.0))
  agg = agg_ref[0] + agg_ref[1]
  o_ref[...] = jnp.maximum(agg * rs[:, None] + b_ref[...], 0.0)


def _finish_tc(aggp, degi_t, b2, N, D, BR):
  return pl.pallas_call(
      _finish_body,
      grid=(N // BR,),
      in_specs=[
          pl.BlockSpec((2, BR, D), lambda i: (0, i, 0)),
          pl.BlockSpec((BR, 2), lambda i: (i, 0)),
          pl.BlockSpec((1, D), lambda i: (0, 0)),
      ],
      out_specs=pl.BlockSpec((BR, D), lambda i: (i, 0)),
      out_shape=jax.ShapeDtypeStruct((N, D), jnp.float32),
  )(aggp, degi_t, b2)


def kernel(h, edge_index, W, b):
  N, D_in = h.shape
  D = W.shape[1]
  E = edge_index.shape[1]
  NW = NC * NS

  CH = -(-E // (NW * EB))
  E_pad = NW * CH * EB
  src = edge_index[0]
  dst = edge_index[1]
  pad = E_pad - E
  sent = jnp.full((pad,), N, jnp.int32)
  src_p = jnp.concatenate([src, sent]).reshape(NW, CH, EB)
  dst_flat = jnp.concatenate([dst, sent])
  dst_p = dst_flat.reshape(NW, CH, EB)

  NHP = -(-(N + 1) // (NS * 8)) * (NS * 8)
  hp = _make_deg_kernel(CH, NHP)(src_p, dst_p).reshape(NC, 2, NHP)
  dego_t = hp[:, 0, :N].T
  degi_t = hp[:, 1, :N].T

  BR = 1000
  feat = _feat_tc(h, W, dego_t, BR)

  NFP = -(-(N + 1) // (NS * 8)) * (NS * 8)
  feat_pad = jnp.zeros((NFP, D), jnp.float32).at[:N].set(feat)

  aggp = _make_agg_kernel(NFP, D, CH)(feat_pad, src_p, dst_p)

  out = _finish_tc(aggp, degi_t, b.reshape(1, D), N, D, BR)
  return out
